# baseline (speedup 1.0000x reference)
import functools

import numpy as np
import jax
import jax.numpy as jnp
from jax import lax
from jax.experimental import pallas as pl
from jax.experimental.pallas import tpu as pltpu
from jax.experimental.pallas import tpu_sc as plsc

_T = 2 ** 19
_L = 16
_NC, _NS = 2, 16
_NW = _NC * _NS
_B = 256
_CH = 512
_CHS = 9
_BN = 2048

_PRIMES = np.array([1, 2654435761, 805459861, 3674653429], dtype=np.uint64)


def _level_params(ndim):
    base, scale = 16.0, 1.3819
    rm1, rm2, m1, m2, m3, isx = [], [], [], [], [], []
    for l in range(_L):
        res = max(int(np.ceil(base * (scale ** l))), 2)
        R = res + 1
        use_hash = float(R) ** ndim > float(_T)
        rm1.append(np.float32(res - 1))
        rm2.append(np.uint32(res - 2))
        if use_hash:
            mm = [np.uint32(_PRIMES[d]) for d in range(1, ndim)]
        else:
            strides = (np.cumprod(
                np.concatenate([[1], np.full(ndim - 1, R)]).astype(np.uint64)
            ) % (2 ** 32)).astype(np.uint32)
            mm = [strides[d] for d in range(1, ndim)]
        m1.append(mm[0])
        m2.append(mm[1])
        m3.append(mm[2] if ndim == 4 else np.uint32(0))
        isx.append(np.uint32(1 if use_hash else 0))
    farr = np.repeat(np.array(rm1, np.float32)[:, None], 16, axis=1).reshape(-1)

    def spl(v):
        return np.repeat(np.array(v, np.uint32).view(np.int32)[:, None], 16, 1)

    iarr = np.stack([spl(rm2), spl(m1), spl(m2), spl(m3), spl(isx)],
                    axis=1).reshape(-1)
    return farr, iarr


def _sc_body(N, nblk, xt, sp2, tp2, mp2, lvsf, lvsi, lvtf, lvti,
             sfeat, tfeat, mout,
             xv, pfs, pis, pft, pit, idxb, rows, mrows, wb,
             sacc, tacc, macc, sem):
    wid = lax.axis_index("s") * _NC + lax.axis_index("c")
    iota16 = lax.iota(jnp.int32, 16)
    iota32x = iota16 * 32
    z16 = jnp.zeros((16,), jnp.int32)
    o16 = z16 + 1

    pltpu.sync_copy(lvsf, pfs)
    pltpu.sync_copy(lvsi, pis)
    pltpu.sync_copy(lvtf, pft)
    pltpu.sync_copy(lvti, pit)

    def corners_pass(ndim, ncor, gb, rm1, rm2, ms, isx, lT):
        q0, q1, u = [], [], []
        for d in range(ndim):
            xd = xv[pl.ds(d * _B + gb, 16)]
            posd = xd * rm1
            pid = jnp.maximum(jnp.minimum(posd.astype(jnp.int32), rm2), 0)
            frd = posd - pid.astype(jnp.float32)
            if d == 0:
                qa = pid
                qb = pid + 1
            else:
                qa = pid * ms[d - 1]
                qb = qa + ms[d - 1]
            q0.append(qa)
            q1.append(qb)
            u.append((1.0 - frd, frd))
        p01 = [u[0][b0] * u[1][b1] for b1 in range(2) for b0 in range(2)]
        if ndim == 4:
            p23 = [u[2][b2] * u[3][b3] for b3 in range(2) for b2 in range(2)]
        for corner in range(ncor):
            bs_ = [(corner >> d) & 1 for d in range(ndim)]
            qs = [q1[d] if bs_[d] else q0[d] for d in range(ndim)]
            s_add = qs[0] + qs[1] + qs[2]
            s_xor = qs[0] ^ qs[1] ^ qs[2]
            if ndim == 4:
                s_add = s_add + qs[3]
                s_xor = s_xor ^ qs[3]
            tt = jnp.where(isx, s_xor, s_add)
            idx = (tt & (_T - 1)) + lT
            off = corner * _B + gb
            idxb[pl.ds(off, 16)] = idx
            if ndim == 3:
                w = p01[bs_[1] * 2 + bs_[0]] * u[2][bs_[2]]
            else:
                w = p01[bs_[1] * 2 + bs_[0]] * p23[bs_[3] * 2 + bs_[2]]
            wb[pl.ds(off, 16)] = w

    def fire_drain(tb, dst, nch):
        def fire(c, c_):
            pltpu.async_copy(tb.at[idxb.at[pl.ds(c * _CH, _CH)]],
                             dst.at[c], sem)
            return c_
        lax.fori_loop(0, nch, fire, 0)

        def drain(c, c_):
            pltpu.make_async_copy(tb.at[idxb.at[pl.ds(c * _CH, _CH)]],
                                  dst.at[c], sem).wait()
            return c_
        lax.fori_loop(0, nch, drain, 0)

    def run_encode(ndim, ncor, tb2, pfv, piv, acc):
        nch = ncor * _B // _CH

        def lvl(l, c2_):
            rm1 = pfv[pl.ds(l * 16, 16)]
            pb = l * 5 * 16
            rm2 = piv[pl.ds(pb, 16)]
            m1 = piv[pl.ds(pb + 16, 16)]
            m2 = piv[pl.ds(pb + 32, 16)]
            m3 = piv[pl.ds(pb + 48, 16)]
            isx = piv[pl.ds(pb + 64, 16)] != 0
            ms = [m1, m2, m3]
            lT = l * _T

            def grp(g, c_):
                corners_pass(ndim, ncor, g * 16, rm1, rm2, ms, isx, lT)
                return c_
            lax.fori_loop(0, _B // 16, grp, 0)

            fire_drain(tb2, rows, nch)

            def red(g, c_):
                gb = g * 16
                a0 = jnp.zeros((16,), jnp.float32)
                a1 = jnp.zeros((16,), jnp.float32)
                for corner in range(ncor):
                    off = corner * _B + gb
                    w = wb[pl.ds(off, 16)]
                    vpos = off + iota16
                    cc = lax.shift_right_logical(vpos, _CHS)
                    rr = vpos & (_CH - 1)
                    a0 = a0 + w * plsc.load_gather(rows, [cc, rr, z16])
                    a1 = a1 + w * plsc.load_gather(rows, [cc, rr, o16])
                sidx = gb * 32 + 2 * l + iota32x
                plsc.store_scatter(acc, [sidx], a0)
                plsc.store_scatter(acc, [sidx + 1], a1)
                return c_
            lax.fori_loop(0, _B // 16, red, 0)
            return c2_
        lax.fori_loop(0, _L, lvl, 0)

    def run_mask():
        def grp(g, c_):
            gb = g * 16
            q0, q1, u = [], [], []
            for d in range(3):
                xd = xv[pl.ds(d * _B + gb, 16)]
                posd = xd * 63.0
                pid = jnp.maximum(jnp.minimum(posd.astype(jnp.int32), 62), 0)
                frd = posd - pid.astype(jnp.float32)
                mstat = [1, 65, 4225][d]
                qa = pid * mstat if d else pid
                q0.append(qa)
                q1.append(qa + mstat)
                u.append((1.0 - frd, frd))
            x3 = xv[pl.ds(3 * _B + gb, 16)]
            pos3 = x3 * 63.0
            pi3 = jnp.maximum(jnp.minimum(pos3.astype(jnp.int32), 62), 0)
            fr3 = pos3 - pi3.astype(jnp.float32)
            r3 = jnp.where(fr3 == 0.5, 0, (fr3 + 0.5).astype(jnp.int32))
            q3 = (pi3 + r3) * 274625
            p01 = [u[0][b0] * u[1][b1] for b1 in range(2) for b0 in range(2)]
            for corner in range(8):
                bs_ = [(corner >> d) & 1 for d in range(3)]
                tt = ((q1[0] if bs_[0] else q0[0])
                      + (q1[1] if bs_[1] else q0[1])
                      + (q1[2] if bs_[2] else q0[2]) + q3)
                off = corner * _B + gb
                idxb[pl.ds(off, 16)] = tt & (_T - 1)
                wb[pl.ds(off, 16)] = p01[bs_[1] * 2 + bs_[0]] * u[2][bs_[2]]
            return c_
        lax.fori_loop(0, _B // 16, grp, 0)

        fire_drain(mp2, mrows, 8 * _B // _CH)

        def red(g, c_):
            gb = g * 16
            a = jnp.zeros((16,), jnp.float32)
            for corner in range(8):
                off = corner * _B + gb
                vpos = off + iota16
                cc = lax.shift_right_logical(vpos, _CHS)
                rr = vpos & (_CH - 1)
                a = a + wb[pl.ds(off, 16)] * plsc.load_gather(mrows, [cc, rr, z16])
            macc[pl.ds(gb, 16)] = a
            return c_
        lax.fori_loop(0, _B // 16, red, 0)

    def blk_body(blk, c_):
        base = wid * (nblk * _B) + blk * _B
        for d in range(4):
            pltpu.sync_copy(xt.at[pl.ds(d * N + base, _B)],
                            xv.at[pl.ds(d * _B, _B)])
        run_encode(3, 8, sp2, pfs, pis, sacc)
        run_encode(4, 16, tp2, pft, pit, tacc)
        run_mask()
        pltpu.sync_copy(sacc, sfeat.at[pl.ds(base * 32, _B * 32)])
        pltpu.sync_copy(tacc, tfeat.at[pl.ds(base * 32, _B * 32)])
        pltpu.sync_copy(macc, mout.at[pl.ds(base, _B)])
        return c_

    lax.fori_loop(0, nblk, blk_body, 0)


def _tc_body(s_ref, t_ref, m_ref, ws_ref, bs_ref, wt_ref, bt_ref,
             out_ref, spc_ref):
    dn = (((1,), (0,)), ((), ()))
    spc = lax.dot_general(s_ref[...], ws_ref[...], dn,
                          preferred_element_type=jnp.float32,
                          precision=lax.Precision.HIGHEST) + bs_ref[...]
    tmc = lax.dot_general(t_ref[...], wt_ref[...], dn,
                          preferred_element_type=jnp.float32,
                          precision=lax.Precision.HIGHEST) + bt_ref[...]
    m = jax.nn.sigmoid(m_ref[...])
    out_ref[...] = spc * m + (1.0 - m) * tmc
    spc_ref[...] = spc


@jax.jit
def _impl(x, spatial_params, temporal_params, mask_params, W_s, b_s, W_t, b_t):
    N = x.shape[0]
    L, T, F = spatial_params.shape
    xt = x.T.reshape(-1)
    sp2 = spatial_params.reshape(L * T, F)
    tp2 = temporal_params.reshape(L * T, F)
    mp2 = mask_params.reshape(T, 1)
    fs, is_ = _level_params(3)
    ft, it_ = _level_params(4)
    lvsf = jnp.asarray(fs)
    lvsi = jnp.asarray(is_)
    lvtf = jnp.asarray(ft)
    lvti = jnp.asarray(it_)

    nblk = N // (_NW * _B)
    mesh = plsc.VectorSubcoreMesh(core_axis_name="c", subcore_axis_name="s",
                                  num_cores=_NC, num_subcores=_NS)
    sc = pl.kernel(
        functools.partial(_sc_body, N, nblk),
        out_type=[jax.ShapeDtypeStruct((N * 32,), jnp.float32),
                  jax.ShapeDtypeStruct((N * 32,), jnp.float32),
                  jax.ShapeDtypeStruct((N,), jnp.float32)],
        mesh=mesh,
        compiler_params=pltpu.CompilerParams(needs_layout_passes=False, use_tc_tiling_on_sc=False),
        scratch_types=[
            pltpu.VMEM((4 * _B,), jnp.float32),
            pltpu.VMEM((_L * 16,), jnp.float32),
            pltpu.VMEM((_L * 5 * 16,), jnp.int32),
            pltpu.VMEM((_L * 16,), jnp.float32),
            pltpu.VMEM((_L * 5 * 16,), jnp.int32),
            pltpu.VMEM((16 * _B,), jnp.int32),
            pltpu.VMEM((16 * _B // _CH, _CH, 2), jnp.float32),
            pltpu.VMEM((8 * _B // _CH, _CH, 1), jnp.float32),
            pltpu.VMEM((16 * _B,), jnp.float32),
            pltpu.VMEM((_B * 32,), jnp.float32),
            pltpu.VMEM((_B * 32,), jnp.float32),
            pltpu.VMEM((_B,), jnp.float32),
            pltpu.SemaphoreType.DMA,
        ],
    )
    sfeat, tfeat, mv = sc(xt, sp2, tp2, mp2, lvsf, lvsi, lvtf, lvti)
    sfeat = sfeat.reshape(N, 32)
    tfeat = tfeat.reshape(N, 32)

    grid = (N // _BN,)
    out, spc = pl.pallas_call(
        _tc_body,
        grid=grid,
        in_specs=[
            pl.BlockSpec((_BN, 32), lambda i: (i, 0)),
            pl.BlockSpec((_BN, 32), lambda i: (i, 0)),
            pl.BlockSpec((_BN, 1), lambda i: (i, 0)),
            pl.BlockSpec((32, 16), lambda i: (0, 0)),
            pl.BlockSpec((1, 16), lambda i: (0, 0)),
            pl.BlockSpec((32, 16), lambda i: (0, 0)),
            pl.BlockSpec((1, 16), lambda i: (0, 0)),
        ],
        out_specs=[pl.BlockSpec((_BN, 16), lambda i: (i, 0)),
                   pl.BlockSpec((_BN, 16), lambda i: (i, 0))],
        out_shape=[jax.ShapeDtypeStruct((N, 16), jnp.float32),
                   jax.ShapeDtypeStruct((N, 16), jnp.float32)],
    )(sfeat, tfeat, mv.reshape(N, 1), W_s, b_s.reshape(1, 16),
      W_t, b_t.reshape(1, 16))
    return out, spc


def kernel(x, spatial_params, temporal_params, mask_params, W_s, b_s, W_t, b_t):
    return _impl(x, spatial_params, temporal_params, mask_params,
                 W_s, b_s, W_t, b_t)


# speedup vs baseline: 5.8306x; 5.8306x over previous
import functools

import numpy as np
import jax
import jax.numpy as jnp
from jax import lax
from jax.experimental import pallas as pl
from jax.experimental.pallas import tpu as pltpu
from jax.experimental.pallas import tpu_sc as plsc

_T = 2 ** 19
_L = 16
_NC, _NS = 2, 16
_NW = _NC * _NS
_B = 512
_CH = 512
_BN = 2048

_PRIMES = np.array([1, 2654435761, 805459861, 3674653429], dtype=np.uint64)


def _level_params(ndim):
    base, scale = 16.0, 1.3819
    rm1, rm2, m1, m2, m3, isx = [], [], [], [], [], []
    for l in range(_L):
        res = max(int(np.ceil(base * (scale ** l))), 2)
        R = res + 1
        use_hash = float(R) ** ndim > float(_T)
        rm1.append(np.float32(res - 1))
        rm2.append(np.uint32(res - 2))
        if use_hash:
            mm = [np.uint32(_PRIMES[d]) for d in range(1, ndim)]
        else:
            strides = (np.cumprod(
                np.concatenate([[1], np.full(ndim - 1, R)]).astype(np.uint64)
            ) % (2 ** 32)).astype(np.uint32)
            mm = [strides[d] for d in range(1, ndim)]
        m1.append(mm[0])
        m2.append(mm[1])
        m3.append(mm[2] if ndim == 4 else np.uint32(0))
        isx.append(np.uint32(1 if use_hash else 0))
    farr = np.repeat(np.array(rm1, np.float32)[:, None], 16, axis=1).reshape(-1)

    def spl(v):
        return np.repeat(np.array(v, np.uint32).view(np.int32)[:, None], 16, 1)

    iarr = np.stack([spl(rm2), spl(m1), spl(m2), spl(m3), spl(isx)],
                    axis=1).reshape(-1)
    return farr, iarr


def _sc_body(N, nblk, xt, sp0, sp1, tp0, tp1, mpf, lvsf, lvsi, lvtf, lvti,
             sfeat, tfeat, mout,
             xv, pfs, pis, pft, pit, idxb, r0, r1, mr, wb,
             sacc, tacc, macc, sem):
    wid = lax.axis_index("s") * _NC + lax.axis_index("c")
    iota16 = lax.iota(jnp.int32, 16)
    iota32x = iota16 * 32
    nbuf = _B * 16

    pltpu.sync_copy(lvsf, pfs)
    pltpu.sync_copy(lvsi, pis)
    pltpu.sync_copy(lvtf, pft)
    pltpu.sync_copy(lvti, pit)

    def corners_pass(ndim, ncor, bb, gb, rm1, rm2, ms, isx, lT):
        q0, q1, u = [], [], []
        for d in range(ndim):
            xd = xv[pl.ds(d * _B + gb, 16)]
            posd = xd * rm1
            pid = jnp.maximum(jnp.minimum(posd.astype(jnp.int32), rm2), 0)
            frd = posd - pid.astype(jnp.float32)
            if d == 0:
                qa = pid
                qb = pid + 1
            else:
                qa = pid * ms[d - 1]
                qb = qa + ms[d - 1]
            q0.append(qa)
            q1.append(qb)
            u.append((1.0 - frd, frd))
        p01 = [u[0][b0] * u[1][b1] for b1 in range(2) for b0 in range(2)]
        if ndim == 4:
            p23 = [u[2][b2] * u[3][b3] for b3 in range(2) for b2 in range(2)]
        for corner in range(ncor):
            bs_ = [(corner >> d) & 1 for d in range(ndim)]
            qs = [q1[d] if bs_[d] else q0[d] for d in range(ndim)]
            s_add = qs[0] + qs[1] + qs[2]
            s_xor = qs[0] ^ qs[1] ^ qs[2]
            if ndim == 4:
                s_add = s_add + qs[3]
                s_xor = s_xor ^ qs[3]
            tt = jnp.where(isx, s_xor, s_add)
            idx = (tt & (_T - 1)) + lT
            off = bb + corner * _B + gb
            idxb[pl.ds(off, 16)] = idx
            if ndim == 3:
                w = p01[bs_[1] * 2 + bs_[0]] * u[2][bs_[2]]
            else:
                w = p01[bs_[1] * 2 + bs_[0]] * p23[bs_[3] * 2 + bs_[2]]
            wb[pl.ds(off, 16)] = w

    def fire(tables, dsts, bb, nch):
        def f(c, c_):
            for tb, dst in zip(tables, dsts):
                pltpu.async_copy(tb.at[idxb.at[pl.ds(bb + c * _CH, _CH)]],
                                 dst.at[pl.ds(bb + c * _CH, _CH)], sem)
            return c_
        lax.fori_loop(0, nch, f, 0)

    def drain(tables, dsts, bb, nch):
        def f(c, c_):
            for tb, dst in zip(tables, dsts):
                pltpu.make_async_copy(tb.at[idxb.at[pl.ds(bb + c * _CH, _CH)]],
                                      dst.at[pl.ds(bb + c * _CH, _CH)],
                                      sem).wait()
            return c_
        lax.fori_loop(0, nch, f, 0)

    def run_encode(ndim, ncor, t0, t1, pfv, piv, acc):
        nch = ncor * _B // _CH

        def compute_level(l, bb):
            rm1 = pfv[pl.ds(l * 16, 16)]
            pb = l * 5 * 16
            rm2 = piv[pl.ds(pb, 16)]
            m1 = piv[pl.ds(pb + 16, 16)]
            m2 = piv[pl.ds(pb + 32, 16)]
            m3 = piv[pl.ds(pb + 48, 16)]
            isx = piv[pl.ds(pb + 64, 16)] != 0
            ms = [m1, m2, m3]
            lT = l * _T

            def grp(g, c_):
                corners_pass(ndim, ncor, bb, g * 16, rm1, rm2, ms, isx, lT)
                return c_
            lax.fori_loop(0, _B // 16, grp, 0)

        def reduce_level(l, bb):
            def red(g, c_):
                gb = g * 16
                a0 = jnp.zeros((16,), jnp.float32)
                a1 = jnp.zeros((16,), jnp.float32)
                for corner in range(ncor):
                    off = bb + corner * _B + gb
                    w = wb[pl.ds(off, 16)]
                    a0 = a0 + w * r0[pl.ds(off, 16)]
                    a1 = a1 + w * r1[pl.ds(off, 16)]
                sidx = gb * 32 + 2 * l + iota32x
                plsc.store_scatter(acc, [sidx], a0)
                plsc.store_scatter(acc, [sidx + 1], a1)
                return c_
            lax.fori_loop(0, _B // 16, red, 0)

        compute_level(0, 0)
        fire([t0, t1], [r0, r1], 0, nch)

        def lvl(l, c2_):
            bb = (l & 1) * nbuf
            nb = ((l + 1) & 1) * nbuf

            @pl.when(l < _L - 1)
            def _():
                compute_level(l + 1, nb)
                fire([t0, t1], [r0, r1], nb, nch)

            drain([t0, t1], [r0, r1], bb, nch)
            reduce_level(l, bb)
            return c2_
        lax.fori_loop(0, _L, lvl, 0)

    def run_mask():
        def grp(g, c_):
            gb = g * 16
            q0, q1, u = [], [], []
            for d in range(3):
                xd = xv[pl.ds(d * _B + gb, 16)]
                posd = xd * 63.0
                pid = jnp.maximum(jnp.minimum(posd.astype(jnp.int32), 62), 0)
                frd = posd - pid.astype(jnp.float32)
                mstat = [1, 65, 4225][d]
                qa = pid * mstat if d else pid
                q0.append(qa)
                q1.append(qa + mstat)
                u.append((1.0 - frd, frd))
            x3 = xv[pl.ds(3 * _B + gb, 16)]
            pos3 = x3 * 63.0
            pi3 = jnp.maximum(jnp.minimum(pos3.astype(jnp.int32), 62), 0)
            fr3 = pos3 - pi3.astype(jnp.float32)
            r3 = jnp.where(fr3 == 0.5, 0, (fr3 + 0.5).astype(jnp.int32))
            q3 = (pi3 + r3) * 274625
            p01 = [u[0][b0] * u[1][b1] for b1 in range(2) for b0 in range(2)]
            for corner in range(8):
                bs_ = [(corner >> d) & 1 for d in range(3)]
                tt = ((q1[0] if bs_[0] else q0[0])
                      + (q1[1] if bs_[1] else q0[1])
                      + (q1[2] if bs_[2] else q0[2]) + q3)
                off = corner * _B + gb
                idxb[pl.ds(off, 16)] = tt & (_T - 1)
                wb[pl.ds(off, 16)] = p01[bs_[1] * 2 + bs_[0]] * u[2][bs_[2]]
            return c_
        lax.fori_loop(0, _B // 16, grp, 0)

        nchm = 8 * _B // _CH
        fire([mpf], [mr], 0, nchm)
        drain([mpf], [mr], 0, nchm)

        def red(g, c_):
            gb = g * 16
            a = jnp.zeros((16,), jnp.float32)
            for corner in range(8):
                off = corner * _B + gb
                a = a + wb[pl.ds(off, 16)] * mr[pl.ds(off, 16)]
            macc[pl.ds(gb, 16)] = a
            return c_
        lax.fori_loop(0, _B // 16, red, 0)

    def blk_body(blk, c_):
        base = wid * (nblk * _B) + blk * _B
        for d in range(4):
            pltpu.sync_copy(xt.at[pl.ds(d * N + base, _B)],
                            xv.at[pl.ds(d * _B, _B)])
        run_encode(3, 8, sp0, sp1, pfs, pis, sacc)
        run_encode(4, 16, tp0, tp1, pft, pit, tacc)
        run_mask()
        pltpu.sync_copy(sacc, sfeat.at[pl.ds(base * 32, _B * 32)])
        pltpu.sync_copy(tacc, tfeat.at[pl.ds(base * 32, _B * 32)])
        pltpu.sync_copy(macc, mout.at[pl.ds(base, _B)])
        return c_

    lax.fori_loop(0, nblk, blk_body, 0)


def _tc_body(s_ref, t_ref, m_ref, ws_ref, bs_ref, wt_ref, bt_ref,
             out_ref, spc_ref):
    dn = (((1,), (0,)), ((), ()))
    spc = lax.dot_general(s_ref[...], ws_ref[...], dn,
                          preferred_element_type=jnp.float32,
                          precision=lax.Precision.HIGHEST) + bs_ref[...]
    tmc = lax.dot_general(t_ref[...], wt_ref[...], dn,
                          preferred_element_type=jnp.float32,
                          precision=lax.Precision.HIGHEST) + bt_ref[...]
    m = jax.nn.sigmoid(m_ref[...])
    out_ref[...] = spc * m + (1.0 - m) * tmc
    spc_ref[...] = spc


@jax.jit
def _impl(x, spatial_params, temporal_params, mask_params, W_s, b_s, W_t, b_t):
    N = x.shape[0]
    L, T, F = spatial_params.shape
    xt = x.T.reshape(-1)
    sp0 = spatial_params[:, :, 0].reshape(-1)
    sp1 = spatial_params[:, :, 1].reshape(-1)
    tp0 = temporal_params[:, :, 0].reshape(-1)
    tp1 = temporal_params[:, :, 1].reshape(-1)
    mpf = mask_params.reshape(-1)
    fs, is_ = _level_params(3)
    ft, it_ = _level_params(4)
    lvsf = jnp.asarray(fs)
    lvsi = jnp.asarray(is_)
    lvtf = jnp.asarray(ft)
    lvti = jnp.asarray(it_)

    nblk = N // (_NW * _B)
    mesh = plsc.VectorSubcoreMesh(core_axis_name="c", subcore_axis_name="s",
                                  num_cores=_NC, num_subcores=_NS)
    sc = pl.kernel(
        functools.partial(_sc_body, N, nblk),
        out_type=[jax.ShapeDtypeStruct((N * 32,), jnp.float32),
                  jax.ShapeDtypeStruct((N * 32,), jnp.float32),
                  jax.ShapeDtypeStruct((N,), jnp.float32)],
        mesh=mesh,
        compiler_params=pltpu.CompilerParams(needs_layout_passes=False),
        scratch_types=[
            pltpu.VMEM((4 * _B,), jnp.float32),
            pltpu.VMEM((_L * 16,), jnp.float32),
            pltpu.VMEM((_L * 5 * 16,), jnp.int32),
            pltpu.VMEM((_L * 16,), jnp.float32),
            pltpu.VMEM((_L * 5 * 16,), jnp.int32),
            pltpu.VMEM((2 * 16 * _B,), jnp.int32),
            pltpu.VMEM((2 * 16 * _B,), jnp.float32),
            pltpu.VMEM((2 * 16 * _B,), jnp.float32),
            pltpu.VMEM((8 * _B,), jnp.float32),
            pltpu.VMEM((2 * 16 * _B,), jnp.float32),
            pltpu.VMEM((_B * 32,), jnp.float32),
            pltpu.VMEM((_B * 32,), jnp.float32),
            pltpu.VMEM((_B,), jnp.float32),
            pltpu.SemaphoreType.DMA,
        ],
    )
    sfeat, tfeat, mv = sc(xt, sp0, sp1, tp0, tp1, mpf,
                          lvsf, lvsi, lvtf, lvti)
    sfeat = sfeat.reshape(N, 32)
    tfeat = tfeat.reshape(N, 32)

    grid = (N // _BN,)
    out, spc = pl.pallas_call(
        _tc_body,
        grid=grid,
        in_specs=[
            pl.BlockSpec((_BN, 32), lambda i: (i, 0)),
            pl.BlockSpec((_BN, 32), lambda i: (i, 0)),
            pl.BlockSpec((_BN, 1), lambda i: (i, 0)),
            pl.BlockSpec((32, 16), lambda i: (0, 0)),
            pl.BlockSpec((1, 16), lambda i: (0, 0)),
            pl.BlockSpec((32, 16), lambda i: (0, 0)),
            pl.BlockSpec((1, 16), lambda i: (0, 0)),
        ],
        out_specs=[pl.BlockSpec((_BN, 16), lambda i: (i, 0)),
                   pl.BlockSpec((_BN, 16), lambda i: (i, 0))],
        out_shape=[jax.ShapeDtypeStruct((N, 16), jnp.float32),
                   jax.ShapeDtypeStruct((N, 16), jnp.float32)],
    )(sfeat, tfeat, mv.reshape(N, 1), W_s, b_s.reshape(1, 16),
      W_t, b_t.reshape(1, 16))
    return out, spc


def kernel(x, spatial_params, temporal_params, mask_params, W_s, b_s, W_t, b_t):
    return _impl(x, spatial_params, temporal_params, mask_params,
                 W_s, b_s, W_t, b_t)
